# baseline (speedup 1.0000x reference)
import functools

import jax
import jax.numpy as jnp
from jax import lax
from jax.experimental import pallas as pl
from jax.experimental.pallas import tpu as pltpu
from jax.experimental.pallas import tpu_sc as plsc

_L = 128
_G = 16
_NW = 32


def _gather_blocks(ids_t, tab_pairs):
    s_len, b_len = ids_t.shape
    _, d2 = tab_pairs.shape
    d = d2 // 2

    mesh = plsc.VectorSubcoreMesh(core_axis_name="c", subcore_axis_name="s")

    @functools.partial(
        pl.kernel,
        out_type=jax.ShapeDtypeStruct((s_len, d, b_len), jnp.float32),
        mesh=mesh,
        compiler_params=pltpu.CompilerParams(
            use_tc_tiling_on_sc=True, needs_layout_passes=False),
        scratch_types=[
            pltpu.VMEM((2, _L), jnp.int32),
            pltpu.VMEM((2, _L), jnp.int32),
            pltpu.VMEM((2, _L), jnp.int32),
            pltpu.VMEM((2, _L, d2), jnp.float32),
            pltpu.VMEM((2, d, _L), jnp.float32),
            pltpu.SemaphoreType.DMA,
            pltpu.SemaphoreType.DMA,
            pltpu.SemaphoreType.DMA,
            pltpu.SemaphoreType.DMA,
            pltpu.SemaphoreType.DMA,
            pltpu.SemaphoreType.DMA,
        ],
    )
    def grab(ids_hbm, tab_hbm, out_hbm, idx_v, pidx_v, poff_v, pair_v,
             out_v, si0, si1, sg0, sg1, so0, so1):
        idx_sems = [si0, si1]
        gat_sems = [sg0, sg1]
        out_sems = [so0, so1]

        nc = lax.axis_size("c")
        wid = lax.axis_index("s") * nc + lax.axis_index("c")
        col0 = wid * _L

        def idx_copy(s, slot):
            return pltpu.make_async_copy(
                ids_hbm.at[s, pl.ds(col0, _L)], idx_v.at[slot],
                idx_sems[slot])

        def gather_copy(slot):
            return pltpu.make_async_copy(
                tab_hbm.at[pidx_v.at[slot]], pair_v.at[slot],
                gat_sems[slot])

        def out_copy(s, slot):
            return pltpu.make_async_copy(
                out_v.at[slot], out_hbm.at[s, :, pl.ds(col0, _L)],
                out_sems[slot])

        def prep(slot):
            for g in range(_L // _G):
                v = idx_v[slot, pl.ds(g * _G, _G)]
                pidx_v[slot, pl.ds(g * _G, _G)] = lax.shift_right_logical(
                    v, 1)
                poff_v[slot, pl.ds(g * _G, _G)] = lax.shift_left(
                    lax.bitwise_and(v, 1), 6)

        def transpose(slot):
            for g in range(_L // _G):
                row = lax.iota(jnp.int32, _G) + g * _G
                col0v = poff_v[slot, pl.ds(g * _G, _G)]

                @pl.loop(0, d, init_carry=col0v, unroll=8)
                def _(k, col):
                    out_v[slot, k, pl.ds(g * _G, _G)] = plsc.load_gather(
                        pair_v.at[slot], [row, col])
                    return col + 1

        def step(s, slot, first, last1, last2):
            if not last1:
                idx_copy(s + 1, 1 - slot).wait()
                prep(1 - slot)
                gather_copy(1 - slot).start()
            if not last2:
                idx_copy(s + 2, slot).start()
            gather_copy(slot).wait()
            if not first:
                out_copy(s - 2, slot).wait()
            transpose(slot)
            out_copy(s, slot).start()

        idx_copy(0, 0).start()
        idx_copy(1, 1).start()
        idx_copy(0, 0).wait()
        prep(0)
        gather_copy(0).start()

        step(0, 0, first=True, last1=False, last2=False)
        step(1, 1, first=True, last1=False, last2=False)

        @pl.loop(2, s_len - 2, step=2)
        def _(s):
            step(s, 0, first=False, last1=False, last2=False)
            step(s + 1, 1, first=False, last1=False, last2=False)

        step(s_len - 2, 0, first=False, last1=False, last2=True)
        step(s_len - 1, 1, first=False, last1=True, last2=True)
        out_copy(s_len - 2, 0).wait()
        out_copy(s_len - 1, 1).wait()

    return grab(ids_t, tab_pairs)


def kernel(input_ids, word_embeddings):
    b, s = input_ids.shape
    v, d = word_embeddings.shape
    ids_t = input_ids.T.astype(jnp.int32)
    tab_pairs = word_embeddings.reshape(v // 2, 2 * d)
    out_t = _gather_blocks(ids_t, tab_pairs)
    return out_t.transpose(2, 0, 1)


# speedup vs baseline: 1.1517x; 1.1517x over previous
import functools

import jax
import jax.numpy as jnp
from jax import lax
from jax.experimental import pallas as pl
from jax.experimental.pallas import tpu as pltpu
from jax.experimental.pallas import tpu_sc as plsc

_L = 128
_G = 16
_NW = 32


def _gather_blocks(ids_t, tab_pairs):
    s_len, b_len = ids_t.shape
    _, d2 = tab_pairs.shape
    d = d2 // 2

    mesh = plsc.VectorSubcoreMesh(core_axis_name="c", subcore_axis_name="s")

    @functools.partial(
        pl.kernel,
        out_type=jax.ShapeDtypeStruct((s_len, d, b_len), jnp.float32),
        mesh=mesh,
        compiler_params=pltpu.CompilerParams(
            use_tc_tiling_on_sc=True, needs_layout_passes=False),
        scratch_types=[
            pltpu.VMEM((2, _L), jnp.int32),
            pltpu.VMEM((2, _L), jnp.int32),
            pltpu.VMEM((2, _L), jnp.int32),
            pltpu.VMEM((2, _L, d2), jnp.float32),
            pltpu.VMEM((2, d, _L), jnp.float32),
            pltpu.SemaphoreType.DMA,
            pltpu.SemaphoreType.DMA,
            pltpu.SemaphoreType.DMA,
            pltpu.SemaphoreType.DMA,
            pltpu.SemaphoreType.DMA,
            pltpu.SemaphoreType.DMA,
        ],
    )
    def grab(ids_hbm, tab_hbm, out_hbm, idx_v, pidx_v, poff_v, pair_v,
             out_v, si0, si1, sg0, sg1, so0, so1):
        idx_sems = [si0, si1]
        gat_sems = [sg0, sg1]
        out_sems = [so0, so1]

        nc = lax.axis_size("c")
        wid = lax.axis_index("s") * nc + lax.axis_index("c")
        col0 = wid * _L

        def idx_copy(s, slot):
            return pltpu.make_async_copy(
                ids_hbm.at[s, pl.ds(col0, _L)], idx_v.at[slot],
                idx_sems[slot])

        def gather_copy(slot):
            return pltpu.make_async_copy(
                tab_hbm.at[pidx_v.at[slot]], pair_v.at[slot],
                gat_sems[slot])

        def out_copy(s, slot):
            return pltpu.make_async_copy(
                out_v.at[slot], out_hbm.at[s, :, pl.ds(col0, _L)],
                out_sems[slot])

        def prep(slot):
            for g in range(_L // _G):
                v = idx_v[slot, pl.ds(g * _G, _G)]
                pidx_v[slot, pl.ds(g * _G, _G)] = lax.shift_right_logical(
                    v, 1)
                poff_v[slot, pl.ds(g * _G, _G)] = lax.shift_left(
                    lax.bitwise_and(v, 1), 6)

        def transpose(slot):
            ng = _L // _G
            rows = [lax.iota(jnp.int32, _G) + g * _G for g in range(ng)]
            cols0 = tuple(poff_v[slot, pl.ds(g * _G, _G)] for g in range(ng))

            @pl.loop(0, d, init_carry=cols0, unroll=2)
            def _(k, cols):
                vecs = [plsc.load_gather(pair_v.at[slot], [rows[g], cols[g]])
                        for g in range(ng)]
                for g in range(ng):
                    out_v[slot, k, pl.ds(g * _G, _G)] = vecs[g]
                return tuple(c + 1 for c in cols)

        def step(s, slot, first, last1, last2):
            if not last1:
                idx_copy(s + 1, 1 - slot).wait()
                prep(1 - slot)
                gather_copy(1 - slot).start()
            if not last2:
                idx_copy(s + 2, slot).start()
            gather_copy(slot).wait()
            if not first:
                out_copy(s - 2, slot).wait()
            transpose(slot)
            out_copy(s, slot).start()

        idx_copy(0, 0).start()
        idx_copy(1, 1).start()
        idx_copy(0, 0).wait()
        prep(0)
        gather_copy(0).start()

        step(0, 0, first=True, last1=False, last2=False)
        step(1, 1, first=True, last1=False, last2=False)

        @pl.loop(2, s_len - 2, step=2)
        def _(s):
            step(s, 0, first=False, last1=False, last2=False)
            step(s + 1, 1, first=False, last1=False, last2=False)

        step(s_len - 2, 0, first=False, last1=False, last2=True)
        step(s_len - 1, 1, first=False, last1=True, last2=True)
        out_copy(s_len - 2, 0).wait()
        out_copy(s_len - 1, 1).wait()

    return grab(ids_t, tab_pairs)


def kernel(input_ids, word_embeddings):
    b, s = input_ids.shape
    v, d = word_embeddings.shape
    ids_t = input_ids.T.astype(jnp.int32)
    tab_pairs = word_embeddings.reshape(v // 2, 2 * d)
    out_t = _gather_blocks(ids_t, tab_pairs)
    return out_t.transpose(2, 0, 1)
